# baseline (speedup 1.0000x reference)
import jax
import jax.numpy as jnp
from jax import lax
from jax.experimental import pallas as pl
from jax.experimental.pallas import tpu as pltpu
from jax.experimental.pallas import tpu_sc as plsc

ENTITY_NUM = 100000
CONCEPT_NUM = 5000
RELATION_NUM = 500
EMB_DIM = 128
B_HLR = 16384
B_INS = 8192
B_SUB = 4096
MARGIN_HLR = 1.0
MARGIN_INS = 0.4
MARGIN_SUB = 0.3

NC = 2
NS = 16
NW = NC * NS
L = 16
CHUNK = 64

HLR_W = B_HLR // NW
INS_W = B_INS // NW
SUB_W = B_SUB // NW
RAD_PAD = 5120
UNROLL = 16


def _vsqrt(x):
    i = plsc.bitcast(x, jnp.int32)
    magic = jnp.full((L,), 0x1FBD1DF5, jnp.int32)
    y = plsc.bitcast((i >> 1) + magic, jnp.float32)
    half = jnp.full((L,), 0.5, jnp.float32)
    for _ in range(3):
        y = half * (y + x / y)
    return jnp.where(x > 0.0, y, 0.0)


def _sc_partials(entity_vec, relation_vec, centers, radii_pad,
                 ih, it, ir, ihn, itn, iep, icp, ien, icn,
                 icip, icjp, icin, icjn):
    mesh = plsc.VectorSubcoreMesh(core_axis_name="c", subcore_axis_name="s")

    def body(ent, rel, cen, rad,
             r_ih, r_it, r_ir, r_ihn, r_itn,
             r_iep, r_icp, r_ien, r_icn,
             r_icip, r_icjp, r_icin, r_icjn,
             out,
             bufs0, bufs1,
             v_ih, v_it, v_ir, v_ihn, v_itn,
             v_iep, v_icp, v_ien, v_icn,
             v_icip, v_icjp, v_icin, v_icjn,
             v_rad, v_out, sem0, sem1):
        cid = lax.axis_index("c")
        sid = lax.axis_index("s")
        wid = sid * NC + cid

        def stage(src, dst, w):
            pltpu.sync_copy(src.at[pl.ds(wid * w, w)], dst)

        stage(r_ih, v_ih, HLR_W)
        stage(r_it, v_it, HLR_W)
        stage(r_ir, v_ir, HLR_W)
        stage(r_ihn, v_ihn, HLR_W)
        stage(r_itn, v_itn, HLR_W)
        stage(r_iep, v_iep, INS_W)
        stage(r_icp, v_icp, INS_W)
        stage(r_ien, v_ien, INS_W)
        stage(r_icn, v_icn, INS_W)
        stage(r_icip, v_icip, SUB_W)
        stage(r_icjp, v_icjp, SUB_W)
        stage(r_icin, v_icin, SUB_W)
        stage(r_icjn, v_icjn, SUB_W)
        pltpu.sync_copy(rad, v_rad)

        iota16 = lax.iota(jnp.int32, L)
        zero = jnp.zeros((L,), jnp.float32)
        bufsets = (bufs0, bufs1)
        sems = (sem0, sem1)

        steps = []
        for j in range(HLR_W // CHUNK):
            steps.append(("hlr", [(ent, v_ih), (ent, v_it), (rel, v_ir),
                                  (ent, v_ihn), (ent, v_itn)], j))
        for j in range(INS_W // CHUNK):
            steps.append(("ins", [(ent, v_iep), (cen, v_icp),
                                  (ent, v_ien), (cen, v_icn)], j))
        for j in range(SUB_W // CHUNK):
            steps.append(("sub", [(cen, v_icip), (cen, v_icjp),
                                  (cen, v_icin), (cen, v_icjn)], j))

        def issue(step, k):
            _, srcs, j = step
            bset = bufsets[k % 2]
            sem = sems[k % 2]
            sl = pl.ds(j * CHUNK, CHUNK)
            return [pltpu.async_copy(tab.at[idx.at[sl]], bset[i], sem)
                    for i, (tab, idx) in enumerate(srcs)]

        def sq_cols(i, dd):
            return (jnp.full((L,), dd, jnp.int32) + i * UNROLL + iota16) \
                & (EMB_DIM - 1)

        def hlr_compute(bset, j, loss):
            ba, bb, bc, bd, be = bset

            def group(g, loss):
                rows = g * L + iota16

                def dbody(i, carry):
                    da, db = carry
                    for dd in range(UNROLL):
                        cols = sq_cols(i, dd)
                        hv = plsc.load_gather(ba, [rows, cols])
                        tv = plsc.load_gather(bb, [rows, cols])
                        rv = plsc.load_gather(bc, [rows, cols])
                        hnv = plsc.load_gather(bd, [rows, cols])
                        tnv = plsc.load_gather(be, [rows, cols])
                        a = hv + rv - tv
                        b = hnv + rv - tnv
                        da = da + a * a
                        db = db + b * b
                    return da, db

                da, db = lax.fori_loop(0, EMB_DIM // UNROLL, dbody,
                                       (zero, zero))
                return loss + jnp.maximum(0.0, MARGIN_HLR + da - db)

            return lax.fori_loop(0, CHUNK // L, group, loss)

        def sqdist_pair(ba, bb, bc, bd, g):
            rows = g * L + iota16

            def dbody(i, carry):
                dp, dn = carry
                for dd in range(UNROLL):
                    cols = sq_cols(i, dd)
                    xv = plsc.load_gather(ba, [rows, cols])
                    yv = plsc.load_gather(bb, [rows, cols])
                    uv = plsc.load_gather(bc, [rows, cols])
                    vv = plsc.load_gather(bd, [rows, cols])
                    a = xv - yv
                    b = uv - vv
                    dp = dp + a * a
                    dn = dn + b * b
                return dp, dn

            return lax.fori_loop(0, EMB_DIM // UNROLL, dbody, (zero, zero))

        def ins_compute(bset, j, loss):
            ba, bb, bc, bd, _ = bset

            def group(g, loss):
                dp, dn = sqdist_pair(ba, bb, bc, bd, g)
                base = pl.ds(j * CHUNK + g * L, L)
                mp = plsc.load_gather(v_rad, [v_icp[base]])
                mn = plsc.load_gather(v_rad, [v_icn[base]])
                fp = _vsqrt(dp) - mp
                fn = _vsqrt(dn) - mn
                return loss + jnp.maximum(0.0, MARGIN_INS + fp - fn)

            return lax.fori_loop(0, CHUNK // L, group, loss)

        def sub_compute(bset, j, loss):
            ba, bb, bc, bd, _ = bset

            def group(g, loss):
                dp, dn = sqdist_pair(ba, bb, bc, bd, g)
                base = pl.ds(j * CHUNK + g * L, L)
                mip = plsc.load_gather(v_rad, [v_icip[base]])
                mjp = plsc.load_gather(v_rad, [v_icjp[base]])
                min_ = plsc.load_gather(v_rad, [v_icin[base]])
                mjn = plsc.load_gather(v_rad, [v_icjn[base]])
                gp = _vsqrt(dp) + mip - mjp
                gn = _vsqrt(dn) + min_ - mjn
                return loss + jnp.maximum(0.0, MARGIN_SUB + gp - gn)

            return lax.fori_loop(0, CHUNK // L, group, loss)

        compute_fns = {"hlr": hlr_compute, "ins": ins_compute,
                       "sub": sub_compute}

        loss = zero
        pending = issue(steps[0], 0)
        for k, step in enumerate(steps):
            if k + 1 < len(steps):
                nxt = issue(steps[k + 1], k + 1)
            for c in pending:
                c.wait()
            kind, _, j = step
            loss = compute_fns[kind](bufsets[k % 2], j, loss)
            if k + 1 < len(steps):
                pending = nxt

        v_out[...] = loss
        pltpu.sync_copy(v_out, out.at[wid])

    return pl.kernel(
        body,
        out_type=jax.ShapeDtypeStruct((NW, L), jnp.float32),
        mesh=mesh,
        compiler_params=pltpu.CompilerParams(needs_layout_passes=False),
        scratch_types=[
            [pltpu.VMEM((CHUNK, EMB_DIM), jnp.float32) for _ in range(5)],
            [pltpu.VMEM((CHUNK, EMB_DIM), jnp.float32) for _ in range(5)],
            pltpu.VMEM((HLR_W,), jnp.int32),
            pltpu.VMEM((HLR_W,), jnp.int32),
            pltpu.VMEM((HLR_W,), jnp.int32),
            pltpu.VMEM((HLR_W,), jnp.int32),
            pltpu.VMEM((HLR_W,), jnp.int32),
            pltpu.VMEM((INS_W,), jnp.int32),
            pltpu.VMEM((INS_W,), jnp.int32),
            pltpu.VMEM((INS_W,), jnp.int32),
            pltpu.VMEM((INS_W,), jnp.int32),
            pltpu.VMEM((SUB_W,), jnp.int32),
            pltpu.VMEM((SUB_W,), jnp.int32),
            pltpu.VMEM((SUB_W,), jnp.int32),
            pltpu.VMEM((SUB_W,), jnp.int32),
            pltpu.VMEM((RAD_PAD,), jnp.float32),
            pltpu.VMEM((L,), jnp.float32),
            pltpu.SemaphoreType.DMA,
            pltpu.SemaphoreType.DMA,
        ],
    )(entity_vec, relation_vec, centers, radii_pad,
      ih, it, ir, ihn, itn, iep, icp, ien, icn, icip, icjp, icin, icjn)


def _sum_body(x_ref, o_ref):
    o_ref[...] = jnp.sum(x_ref[...])[None, None]


def kernel(entity_vec, relation_vec, concept_vec,
           hlr_pos_h, hlr_pos_t, hlr_r, hlr_neg_h, hlr_neg_t,
           ins_e_pos, ins_c_pos, ins_e_neg, ins_c_neg,
           sub_ci_pos, sub_cj_pos, sub_ci_neg, sub_cj_neg):
    centers = concept_vec[:, :EMB_DIM]
    radii_pad = jnp.pad(concept_vec[:, EMB_DIM], (0, RAD_PAD - CONCEPT_NUM))
    i32 = jnp.int32
    partials = _sc_partials(
        entity_vec, relation_vec, centers, radii_pad,
        hlr_pos_h.astype(i32), hlr_pos_t.astype(i32), hlr_r.astype(i32),
        hlr_neg_h.astype(i32), hlr_neg_t.astype(i32),
        ins_e_pos.astype(i32), ins_c_pos.astype(i32),
        ins_e_neg.astype(i32), ins_c_neg.astype(i32),
        sub_ci_pos.astype(i32), sub_cj_pos.astype(i32),
        sub_ci_neg.astype(i32), sub_cj_neg.astype(i32))

    total = pl.pallas_call(
        _sum_body,
        out_shape=jax.ShapeDtypeStruct((1, 1), jnp.float32),
    )(partials)
    return total[0, 0]


# speedup vs baseline: 1.2114x; 1.2114x over previous
import jax
import jax.numpy as jnp
from jax import lax
from jax.experimental import pallas as pl
from jax.experimental.pallas import tpu as pltpu
from jax.experimental.pallas import tpu_sc as plsc

ENTITY_NUM = 100000
CONCEPT_NUM = 5000
RELATION_NUM = 500
EMB_DIM = 128
B_HLR = 16384
B_INS = 8192
B_SUB = 4096
MARGIN_HLR = 1.0
MARGIN_INS = 0.4
MARGIN_SUB = 0.3

NC = 2
NS = 16
NW = NC * NS
L = 16
CHUNK = 64

HLR_W = B_HLR // NW
INS_W = B_INS // NW
SUB_W = B_SUB // NW
RAD_PAD = 5120
UNROLL = 8


def _vsqrt(x):
    i = plsc.bitcast(x, jnp.int32)
    magic = jnp.full((L,), 0x1FBD1DF5, jnp.int32)
    y = plsc.bitcast((i >> 1) + magic, jnp.float32)
    half = jnp.full((L,), 0.5, jnp.float32)
    for _ in range(3):
        y = half * (y + x / y)
    return jnp.where(x > 0.0, y, 0.0)


def _sc_partials(entity_vec, relation_vec, centers, radii_pad,
                 ih, it, ir, ihn, itn, iep, icp, ien, icn,
                 icip, icjp, icin, icjn):
    mesh = plsc.VectorSubcoreMesh(core_axis_name="c", subcore_axis_name="s")

    def body(ent, rel, cen, rad,
             r_ih, r_it, r_ir, r_ihn, r_itn,
             r_iep, r_icp, r_ien, r_icn,
             r_icip, r_icjp, r_icin, r_icjn,
             out,
             bufs0, bufs1,
             v_ih, v_it, v_ir, v_ihn, v_itn,
             v_iep, v_icp, v_ien, v_icn,
             v_icip, v_icjp, v_icin, v_icjn,
             v_rad, v_out, sem0, sem1):
        cid = lax.axis_index("c")
        sid = lax.axis_index("s")
        wid = sid * NC + cid

        stage_list = [
            (r_ih, v_ih, HLR_W), (r_it, v_it, HLR_W), (r_ir, v_ir, HLR_W),
            (r_ihn, v_ihn, HLR_W), (r_itn, v_itn, HLR_W),
            (r_iep, v_iep, INS_W), (r_icp, v_icp, INS_W),
            (r_ien, v_ien, INS_W), (r_icn, v_icn, INS_W),
            (r_icip, v_icip, SUB_W), (r_icjp, v_icjp, SUB_W),
            (r_icin, v_icin, SUB_W), (r_icjn, v_icjn, SUB_W),
        ]
        stage_cps = [pltpu.async_copy(s.at[pl.ds(wid * w, w)], d, sem0)
                     for s, d, w in stage_list]
        stage_cps.append(pltpu.async_copy(rad, v_rad, sem0))
        for c in stage_cps:
            c.wait()

        iota16 = lax.iota(jnp.int32, L)
        zero = jnp.zeros((L,), jnp.float32)
        bufsets = (bufs0, bufs1)
        sems = (sem0, sem1)

        steps = []
        for j in range(HLR_W // CHUNK):
            steps.append(("hlr", [(ent, v_ih), (ent, v_it), (rel, v_ir),
                                  (ent, v_ihn), (ent, v_itn)], j))
        for j in range(INS_W // CHUNK):
            steps.append(("ins", [(ent, v_iep), (cen, v_icp),
                                  (ent, v_ien), (cen, v_icn)], j))
        for j in range(SUB_W // CHUNK):
            steps.append(("sub", [(cen, v_icip), (cen, v_icjp),
                                  (cen, v_icin), (cen, v_icjn)], j))

        def issue(step, k):
            _, srcs, j = step
            bset = bufsets[k % 2]
            sem = sems[k % 2]
            sl = pl.ds(j * CHUNK, CHUNK)
            return [pltpu.async_copy(tab.at[idx.at[sl]], bset[i], sem)
                    for i, (tab, idx) in enumerate(srcs)]

        def sq_cols(i, dd):
            return (jnp.full((L,), dd, jnp.int32) + i * UNROLL + iota16) \
                & (EMB_DIM - 1)

        def hlr_compute(bset, j, loss):
            ba, bb, bc, bd, be = bset

            def group(g, loss):
                rows = g * L + iota16

                def dbody(i, carry):
                    da, db = carry
                    for dd in range(UNROLL):
                        cols = sq_cols(i, dd)
                        hv = plsc.load_gather(ba, [rows, cols])
                        tv = plsc.load_gather(bb, [rows, cols])
                        rv = plsc.load_gather(bc, [rows, cols])
                        hnv = plsc.load_gather(bd, [rows, cols])
                        tnv = plsc.load_gather(be, [rows, cols])
                        a = hv + rv - tv
                        b = hnv + rv - tnv
                        da = da + a * a
                        db = db + b * b
                    return da, db

                da, db = lax.fori_loop(0, EMB_DIM // UNROLL, dbody,
                                       (zero, zero))
                return loss + jnp.maximum(0.0, MARGIN_HLR + da - db)

            return lax.fori_loop(0, CHUNK // L, group, loss)

        def sqdist_pair(ba, bb, bc, bd, g):
            rows = g * L + iota16

            def dbody(i, carry):
                dp, dn = carry
                for dd in range(UNROLL):
                    cols = sq_cols(i, dd)
                    xv = plsc.load_gather(ba, [rows, cols])
                    yv = plsc.load_gather(bb, [rows, cols])
                    uv = plsc.load_gather(bc, [rows, cols])
                    vv = plsc.load_gather(bd, [rows, cols])
                    a = xv - yv
                    b = uv - vv
                    dp = dp + a * a
                    dn = dn + b * b
                return dp, dn

            return lax.fori_loop(0, EMB_DIM // UNROLL, dbody, (zero, zero))

        def ins_compute(bset, j, loss):
            ba, bb, bc, bd, _ = bset

            def group(g, loss):
                dp, dn = sqdist_pair(ba, bb, bc, bd, g)
                base = pl.ds(j * CHUNK + g * L, L)
                mp = plsc.load_gather(v_rad, [v_icp[base]])
                mn = plsc.load_gather(v_rad, [v_icn[base]])
                fp = _vsqrt(dp) - mp
                fn = _vsqrt(dn) - mn
                return loss + jnp.maximum(0.0, MARGIN_INS + fp - fn)

            return lax.fori_loop(0, CHUNK // L, group, loss)

        def sub_compute(bset, j, loss):
            ba, bb, bc, bd, _ = bset

            def group(g, loss):
                dp, dn = sqdist_pair(ba, bb, bc, bd, g)
                base = pl.ds(j * CHUNK + g * L, L)
                mip = plsc.load_gather(v_rad, [v_icip[base]])
                mjp = plsc.load_gather(v_rad, [v_icjp[base]])
                min_ = plsc.load_gather(v_rad, [v_icin[base]])
                mjn = plsc.load_gather(v_rad, [v_icjn[base]])
                gp = _vsqrt(dp) + mip - mjp
                gn = _vsqrt(dn) + min_ - mjn
                return loss + jnp.maximum(0.0, MARGIN_SUB + gp - gn)

            return lax.fori_loop(0, CHUNK // L, group, loss)

        compute_fns = {"hlr": hlr_compute, "ins": ins_compute,
                       "sub": sub_compute}

        loss = zero
        pending = issue(steps[0], 0)
        for k, step in enumerate(steps):
            if k + 1 < len(steps):
                nxt = issue(steps[k + 1], k + 1)
            for c in pending:
                c.wait()
            kind, _, j = step
            loss = compute_fns[kind](bufsets[k % 2], j, loss)
            if k + 1 < len(steps):
                pending = nxt

        v_out[...] = loss
        pltpu.sync_copy(v_out, out.at[wid])

    return pl.kernel(
        body,
        out_type=jax.ShapeDtypeStruct((NW, L), jnp.float32),
        mesh=mesh,
        compiler_params=pltpu.CompilerParams(needs_layout_passes=False),
        scratch_types=[
            [pltpu.VMEM((CHUNK, EMB_DIM), jnp.float32) for _ in range(5)],
            [pltpu.VMEM((CHUNK, EMB_DIM), jnp.float32) for _ in range(5)],
            pltpu.VMEM((HLR_W,), jnp.int32),
            pltpu.VMEM((HLR_W,), jnp.int32),
            pltpu.VMEM((HLR_W,), jnp.int32),
            pltpu.VMEM((HLR_W,), jnp.int32),
            pltpu.VMEM((HLR_W,), jnp.int32),
            pltpu.VMEM((INS_W,), jnp.int32),
            pltpu.VMEM((INS_W,), jnp.int32),
            pltpu.VMEM((INS_W,), jnp.int32),
            pltpu.VMEM((INS_W,), jnp.int32),
            pltpu.VMEM((SUB_W,), jnp.int32),
            pltpu.VMEM((SUB_W,), jnp.int32),
            pltpu.VMEM((SUB_W,), jnp.int32),
            pltpu.VMEM((SUB_W,), jnp.int32),
            pltpu.VMEM((RAD_PAD,), jnp.float32),
            pltpu.VMEM((L,), jnp.float32),
            pltpu.SemaphoreType.DMA,
            pltpu.SemaphoreType.DMA,
        ],
    )(entity_vec, relation_vec, centers, radii_pad,
      ih, it, ir, ihn, itn, iep, icp, ien, icn, icip, icjp, icin, icjn)


def _sum_body(x_ref, o_ref):
    o_ref[...] = jnp.sum(x_ref[...])[None, None]


def kernel(entity_vec, relation_vec, concept_vec,
           hlr_pos_h, hlr_pos_t, hlr_r, hlr_neg_h, hlr_neg_t,
           ins_e_pos, ins_c_pos, ins_e_neg, ins_c_neg,
           sub_ci_pos, sub_cj_pos, sub_ci_neg, sub_cj_neg):
    centers = concept_vec[:, :EMB_DIM]
    radii_pad = jnp.pad(concept_vec[:, EMB_DIM], (0, RAD_PAD - CONCEPT_NUM))
    i32 = jnp.int32
    partials = _sc_partials(
        entity_vec, relation_vec, centers, radii_pad,
        hlr_pos_h.astype(i32), hlr_pos_t.astype(i32), hlr_r.astype(i32),
        hlr_neg_h.astype(i32), hlr_neg_t.astype(i32),
        ins_e_pos.astype(i32), ins_c_pos.astype(i32),
        ins_e_neg.astype(i32), ins_c_neg.astype(i32),
        sub_ci_pos.astype(i32), sub_cj_pos.astype(i32),
        sub_ci_neg.astype(i32), sub_cj_neg.astype(i32))

    total = pl.pallas_call(
        _sum_body,
        out_shape=jax.ShapeDtypeStruct((1, 1), jnp.float32),
    )(partials)
    return total[0, 0]
